# baseline (speedup 1.0000x reference)
import functools

import jax
import jax.numpy as jnp
from jax import lax
from jax.experimental import pallas as pl
from jax.experimental.pallas import tpu as pltpu
from jax.experimental.pallas import tpu_sc as plsc

N_NODES = 1024
DIM = 64
BR = 8


def _score_block(c_ref, w_ref, out_ref):
    w = w_ref[...]
    c = c_ref[...]
    e = jnp.sum(w * c, axis=-1)
    m = jnp.max(e, axis=-1, keepdims=True)
    p = jnp.exp(e - m)
    out_ref[...] = p / jnp.sum(p, axis=-1, keepdims=True)


def _all_scores(context_vector, W, interpret=False):
    c3 = context_vector.reshape(1, 1, DIM)
    n = W.shape[0]
    return pl.pallas_call(
        _score_block,
        grid=(n // BR,),
        in_specs=[
            pl.BlockSpec((1, 1, DIM), lambda i: (0, 0, 0)),
            pl.BlockSpec((BR, N_NODES, DIM), lambda i: (i, 0, 0)),
        ],
        out_specs=pl.BlockSpec((BR, N_NODES), lambda i: (i, 0)),
        out_shape=jax.ShapeDtypeStruct((n, N_NODES), jnp.float32),
        interpret=interpret,
    )(c3, W)


def _make_sc_gather(B, D):
    info = plsc.get_sparse_core_info()
    NC, NS = info.num_cores, info.num_subcores
    NW = NC * NS
    b_per_w = B // NW
    mesh = plsc.VectorSubcoreMesh(core_axis_name="c", subcore_axis_name="s")

    @functools.partial(
        pl.kernel,
        mesh=mesh,
        out_type=jax.ShapeDtypeStruct((B, D), jnp.float32),
        scratch_types=[
            pltpu.VMEM((b_per_w,), jnp.int32),
            pltpu.VMEM((b_per_w, D), jnp.float32),
            pltpu.SemaphoreType.DMA,
        ],
    )
    def gather(table_hbm, idx_hbm, out_hbm, idx_v, rows_v, sem):
        wid = lax.axis_index("s") * NC + lax.axis_index("c")
        base = wid * b_per_w
        pltpu.sync_copy(idx_hbm.at[pl.ds(base, b_per_w)], idx_v)
        pltpu.async_copy(table_hbm.at[idx_v], rows_v, sem).wait()
        pltpu.sync_copy(rows_v, out_hbm.at[pl.ds(base, b_per_w)])

    return gather


def kernel(node_indices, context_vector, W):
    scores = _all_scores(context_vector, W)
    idx = node_indices.astype(jnp.int32)
    gather = _make_sc_gather(node_indices.shape[0], N_NODES)
    return gather(scores, idx)


# speedup vs baseline: 1.1901x; 1.1901x over previous
import functools

import jax
import jax.numpy as jnp
from jax import lax
from jax.experimental import pallas as pl
from jax.experimental.pallas import tpu as pltpu
from jax.experimental.pallas import tpu_sc as plsc

N_NODES = 1024
DIM = 64
BR = 32
NG = 128
GPR = N_NODES // NG
K = NG * DIM


def _score_block(r_ref, x_ref, out_ref):
    e = jnp.dot(x_ref[...], r_ref[...], preferred_element_type=jnp.float32)
    e = e.reshape(BR, GPR, NG)
    m = jnp.max(e, axis=(1, 2), keepdims=True)
    p = jnp.exp(e - m)
    out_ref[...] = p / jnp.sum(p, axis=(1, 2), keepdims=True)


def _all_scores(context_vector, W, interpret=False):
    n = W.shape[0]
    x = W.reshape(n * GPR, K)
    r = jnp.kron(jnp.eye(NG, dtype=jnp.float32), context_vector[:, None])
    s3 = pl.pallas_call(
        _score_block,
        grid=(n // BR,),
        in_specs=[
            pl.BlockSpec((K, NG), lambda i: (0, 0)),
            pl.BlockSpec((BR * GPR, K), lambda i: (i, 0)),
        ],
        out_specs=pl.BlockSpec((BR, GPR, NG), lambda i: (i, 0, 0)),
        out_shape=jax.ShapeDtypeStruct((n, GPR, NG), jnp.float32),
        interpret=interpret,
    )(r, x)
    return s3.reshape(n, N_NODES)


def _make_sc_gather(B, D):
    info = plsc.get_sparse_core_info()
    NC, NS = info.num_cores, info.num_subcores
    NW = NC * NS
    b_per_w = B // NW
    mesh = plsc.VectorSubcoreMesh(core_axis_name="c", subcore_axis_name="s")

    @functools.partial(
        pl.kernel,
        mesh=mesh,
        out_type=jax.ShapeDtypeStruct((B, D), jnp.float32),
        scratch_types=[
            pltpu.VMEM((b_per_w,), jnp.int32),
            pltpu.VMEM((b_per_w, D), jnp.float32),
            pltpu.SemaphoreType.DMA,
        ],
    )
    def gather(table_hbm, idx_hbm, out_hbm, idx_v, rows_v, sem):
        wid = lax.axis_index("s") * NC + lax.axis_index("c")
        base = wid * b_per_w
        pltpu.sync_copy(idx_hbm.at[pl.ds(base, b_per_w)], idx_v)
        pltpu.async_copy(table_hbm.at[idx_v], rows_v, sem).wait()
        pltpu.sync_copy(rows_v, out_hbm.at[pl.ds(base, b_per_w)])

    return gather


def kernel(node_indices, context_vector, W):
    scores = _all_scores(context_vector, W)
    idx = node_indices.astype(jnp.int32)
    gather = _make_sc_gather(node_indices.shape[0], N_NODES)
    return gather(scores, idx)
